# baseline (speedup 1.0000x reference)
import functools

import jax
import jax.numpy as jnp
from jax import lax
from jax.experimental import pallas as pl
from jax.experimental.pallas import tpu as pltpu
from jax.experimental.pallas import tpu_sc as plsc

B, S, D = 4, 2048, 768
E, K = 16, 2
DH = int(D * 4 * 2 / 3)
N = B * S
CAP = int(N * K / E * 1.25)
THRESH = 0.2

TBLK = 512
NTB = N // TBLK
FTILE = 256
NROWS = E * CAP + FTILE
TRASH = E * CAP

NW = 32
TPW = N // NW
DCH = 128
CCH = 64


def _router_body(x_ref, wg_ref, meta_ref, stats_ref, w0s_ref, w1s_ref, cnt_ref):
    t = pl.program_id(0)

    @pl.when(t == 0)
    def _init():
        cnt_ref[...] = jnp.zeros_like(cnt_ref)
        stats_ref[...] = jnp.zeros_like(stats_ref)

    xb = x_ref[...]
    logits = jnp.dot(xb, wg_ref[...], preferred_element_type=jnp.float32)
    m = jnp.max(logits, axis=1, keepdims=True)
    ex = jnp.exp(logits - m)
    sumex = jnp.sum(ex, axis=1, keepdims=True)
    lse = jnp.log(sumex) + m
    probs = ex / sumex

    lane = lax.broadcasted_iota(jnp.int32, (TBLK, E), 1)
    g0 = jnp.max(probs, axis=1, keepdims=True)
    i0 = jnp.min(jnp.where(probs == g0, lane, E), axis=1, keepdims=True)
    c0 = lane == i0
    probs2 = jnp.where(c0, -jnp.inf, probs)
    g1 = jnp.max(probs2, axis=1, keepdims=True)
    i1 = jnp.min(jnp.where(probs2 == g1, lane, E), axis=1, keepdims=True)
    c1 = lane == i1
    g1k = jnp.where(g1 > THRESH, g1, 0.0)

    c0f = c0.astype(jnp.float32)
    c1f = c1.astype(jnp.float32)
    csum = c0f + c1f
    row = lax.broadcasted_iota(jnp.int32, (TBLK, TBLK), 0)
    col = lax.broadcasted_iota(jnp.int32, (TBLK, TBLK), 1)
    tril = (col < row).astype(jnp.float32)
    scum = jnp.dot(tril, csum, preferred_element_type=jnp.float32)
    base = scum + cnt_ref[0:1, :]
    pos0 = jnp.sum(jnp.where(c0, base, 0.0), axis=1, keepdims=True)
    pos1 = jnp.sum(jnp.where(c1, base, 0.0), axis=1, keepdims=True)

    keep0 = pos0 < CAP
    keep1 = pos1 < CAP
    i0f = i0.astype(jnp.float32)
    i1f = i1.astype(jnp.float32)
    src0 = i0f * CAP + jnp.where(keep0, pos0, 0.0)
    src1 = i1f * CAP + jnp.where(keep1, pos1, 0.0)
    w0 = g0 * keep0.astype(jnp.float32)
    w1 = g1k * keep1.astype(jnp.float32)
    dst0 = jnp.where(keep0, i0f * CAP + pos0, float(TRASH))
    dst1 = jnp.where(keep1, i1f * CAP + pos1, float(TRASH))

    zero = jnp.zeros_like(pos0)
    meta_ref[...] = jnp.concatenate(
        [src0, src1, w0, w1, dst0, dst1, zero, zero], axis=1)
    w0s_ref[...] = jnp.broadcast_to(w0, (TBLK, E))
    w1s_ref[...] = jnp.broadcast_to(w1, (TBLK, E))

    cnt_ref[0:1, :] += jnp.sum(csum, axis=0, keepdims=True)
    stats_ref[0:1, :] += jnp.sum(probs, axis=0, keepdims=True)
    stats_ref[1:2, :] += jnp.sum(c0f, axis=0, keepdims=True)
    zpart = jnp.sum(lse * lse)
    stats_ref[2:3, :] += jnp.broadcast_to(zpart, (1, E))


_router = pl.pallas_call(
    _router_body,
    grid=(NTB,),
    in_specs=[
        pl.BlockSpec((TBLK, D), lambda i: (i, 0)),
        pl.BlockSpec((D, E), lambda i: (0, 0)),
    ],
    out_specs=[
        pl.BlockSpec((TBLK, 8), lambda i: (i, 0)),
        pl.BlockSpec((8, E), lambda i: (0, 0)),
        pl.BlockSpec((TBLK, 16), lambda i: (i, 0)),
        pl.BlockSpec((TBLK, 16), lambda i: (i, 0)),
    ],
    out_shape=[
        jax.ShapeDtypeStruct((N, 8), jnp.float32),
        jax.ShapeDtypeStruct((8, E), jnp.float32),
        jax.ShapeDtypeStruct((N, 16), jnp.float32),
        jax.ShapeDtypeStruct((N, 16), jnp.float32),
    ],
    scratch_shapes=[pltpu.VMEM((8, E), jnp.float32)],
)


def _dispatch_body(x_hbm, d0_hbm, d1_hbm, buf_hbm, d0v, d1v, rows, sem):
    wid = lax.axis_index("s") * 2 + lax.axis_index("c")
    nch = TPW // DCH
    pltpu.sync_copy(d0_hbm.at[pl.ds(wid * nch, nch)], d0v)
    pltpu.sync_copy(d1_hbm.at[pl.ds(wid * nch, nch)], d1v)
    for j in range(nch):
        pltpu.sync_copy(x_hbm.at[pl.ds(wid * TPW + j * DCH, DCH)], rows)
        pltpu.async_copy(rows, buf_hbm.at[d0v.at[j]], sem).wait()
        pltpu.async_copy(rows, buf_hbm.at[d1v.at[j]], sem).wait()


def _ffn_body(lng_ref, lnb_ref, b1_ref, b2_ref, x_ref, w1_ref, w2_ref, o_ref):
    xb = x_ref[...]
    mu = jnp.mean(xb, axis=1, keepdims=True)
    xc = xb - mu
    var = jnp.mean(xc * xc, axis=1, keepdims=True)
    xn = xc * lax.rsqrt(var + 1e-5)
    hh = xn * lng_ref[0] + lnb_ref[0]
    h1 = jnp.dot(hh.astype(jnp.bfloat16), w1_ref[0].astype(jnp.bfloat16),
                 preferred_element_type=jnp.float32) + b1_ref[0]
    h1 = jnp.where(h1 >= 0, h1, 0.01 * h1)
    o_ref[...] = jnp.dot(h1.astype(jnp.bfloat16), w2_ref[0].astype(jnp.bfloat16),
                         preferred_element_type=jnp.float32) + b2_ref[0]


_ffn = pl.pallas_call(
    _ffn_body,
    grid=(E, CAP // FTILE),
    in_specs=[
        pl.BlockSpec((1, 1, D), lambda e, j: (e, 0, 0)),
        pl.BlockSpec((1, 1, D), lambda e, j: (e, 0, 0)),
        pl.BlockSpec((1, 1, DH), lambda e, j: (e, 0, 0)),
        pl.BlockSpec((1, 1, D), lambda e, j: (e, 0, 0)),
        pl.BlockSpec((FTILE, D), lambda e, j: (e * (CAP // FTILE) + j, 0)),
        pl.BlockSpec((1, D, DH), lambda e, j: (e, 0, 0)),
        pl.BlockSpec((1, DH, D), lambda e, j: (e, 0, 0)),
    ],
    out_specs=pl.BlockSpec((FTILE, D), lambda e, j: (e * (CAP // FTILE) + j, 0)),
    out_shape=jax.ShapeDtypeStruct((E * CAP, D), jnp.float32),
)


def _combine_body(h_hbm, s0_hbm, s1_hbm, w0_hbm, w1_hbm, out_hbm,
                  s0v, s1v, w0v, w1v, r0, r1, sem0, sem1):
    wid = lax.axis_index("s") * 2 + lax.axis_index("c")
    base = wid * TPW
    for c in range(TPW // CCH):
        tb = base + c * CCH
        pltpu.sync_copy(s0_hbm.at[pl.ds(tb, CCH)], s0v)
        pltpu.sync_copy(s1_hbm.at[pl.ds(tb, CCH)], s1v)
        pltpu.sync_copy(w0_hbm.at[pl.ds(tb, CCH)], w0v)
        pltpu.sync_copy(w1_hbm.at[pl.ds(tb, CCH)], w1v)
        cp0 = pltpu.async_copy(h_hbm.at[s0v], r0, sem0)
        cp1 = pltpu.async_copy(h_hbm.at[s1v], r1, sem1)
        cp0.wait()
        cp1.wait()

        def tok_body(t, carry):
            a0 = w0v[t]
            a1 = w1v[t]
            for dd in range(D // 16):
                sl = pl.ds(dd * 16, 16)
                r0[t, sl] = r0[t, sl] * a0 + r1[t, sl] * a1
            return carry

        lax.fori_loop(0, CCH, tok_body, 0)
        pltpu.sync_copy(r0, out_hbm.at[pl.ds(tb, CCH)])


@functools.lru_cache(maxsize=1)
def _sc_kernels():
    mesh = plsc.VectorSubcoreMesh(
        core_axis_name="c", subcore_axis_name="s", num_cores=2, num_subcores=16)
    dispatch = pl.kernel(
        _dispatch_body,
        out_type=jax.ShapeDtypeStruct((NROWS, D), jnp.float32),
        mesh=mesh,
        scratch_types=[
            pltpu.VMEM((TPW // DCH, DCH), jnp.int32),
            pltpu.VMEM((TPW // DCH, DCH), jnp.int32),
            pltpu.VMEM((DCH, D), jnp.float32),
            pltpu.SemaphoreType.DMA,
        ],
    )
    combine = pl.kernel(
        _combine_body,
        out_type=jax.ShapeDtypeStruct((N, D), jnp.float32),
        mesh=mesh,
        scratch_types=[
            pltpu.VMEM((CCH,), jnp.int32),
            pltpu.VMEM((CCH,), jnp.int32),
            pltpu.VMEM((CCH, 16), jnp.float32),
            pltpu.VMEM((CCH, 16), jnp.float32),
            pltpu.VMEM((CCH, D), jnp.float32),
            pltpu.VMEM((CCH, D), jnp.float32),
            pltpu.SemaphoreType.DMA,
            pltpu.SemaphoreType.DMA,
        ],
    )
    return dispatch, combine


def kernel(x, Wg, ln_g, ln_b, W1, b1, W2, b2):
    _dispatch, _combine = _sc_kernels()
    xf = x.reshape(N, D)
    meta, stats, w0s, w1s = _router(xf, Wg)
    src0 = meta[:, 0].astype(jnp.int32)
    src1 = meta[:, 1].astype(jnp.int32)
    d0 = meta[:, 4].astype(jnp.int32).reshape(N // DCH, DCH)
    d1 = meta[:, 5].astype(jnp.int32).reshape(N // DCH, DCH)

    me = stats[0, :] / N
    ce = stats[1, :] / N
    z_loss = stats[2, 0] / N
    balance = E * jnp.sum(me * ce)
    total = 0.01 * balance + 0.001 * z_loss

    buf = _dispatch(xf, d0, d1)
    h = _ffn(ln_g.reshape(E, 1, D), ln_b.reshape(E, 1, D),
             b1.reshape(E, 1, DH), b2.reshape(E, 1, D), buf, W1, W2)
    out = _combine(h, src0, src1, w0s, w1s)
    return out.reshape(B, S, D), total, balance, z_loss
